# baseline (speedup 1.0000x reference)
import functools

import jax
import jax.numpy as jnp
from jax import lax
from jax.experimental import pallas as pl
from jax.experimental.pallas import tpu as pltpu
from jax.experimental.pallas import tpu_sc as plsc

_NC = 2
_NS = 16
_NW = _NC * _NS
_L = 16

_ROWS = 131072
_COLS = 80
_RSC = 8192
_RTC = _ROWS - _RSC
_RPT = _RSC // _NW
_RCHUNK = 128
_G = _RPT // _RCHUNK
_CV = _COLS // _L

_RBLK = 512
_GTC = _RTC // _RBLK

_NBINS = 10
_TRASH = 12
_UCLAMP = 2.31

_LOG1P_C = (
    -0.006151544861495495, 0.03485012799501419, -0.09325294196605682,
    0.16582375764846802, -0.23982678353786469, 0.3315488398075104,
    -0.49983859062194824, 0.9999942779541016, 3.380091939675367e-08,
)
_SIG_C = (
    -5.25261384609621e-06, 0.00013854062126483768, -0.001915045897476375,
    0.020575666800141335, -0.2081817090511322, 2.499974250793457,
)

_mesh = plsc.VectorSubcoreMesh(core_axis_name="c", subcore_axis_name="s")


@functools.partial(
    pl.kernel,
    out_type=jax.ShapeDtypeStruct((_NW * 2 * _NBINS * _L,), jnp.float32),
    mesh=_mesh,
    scratch_types=[
        pltpu.VMEM((_RCHUNK, _COLS), jnp.float32),
        pltpu.VMEM((_RCHUNK, _COLS), jnp.float32),
        pltpu.VMEM((_RCHUNK, _COLS), jnp.int32),
        pltpu.VMEM((_RCHUNK, _COLS), jnp.int32),
        pltpu.VMEM((_RCHUNK, _COLS), jnp.int32),
        pltpu.VMEM((_RCHUNK, _COLS), jnp.int32),
        pltpu.VMEM((2 * _NBINS * _L,), jnp.float32),
        pltpu.SemaphoreType.DMA,
        pltpu.SemaphoreType.DMA,
    ],
    compiler_params=pltpu.CompilerParams(
        needs_layout_passes=False, use_tc_tiling_on_sc=True),
)
def _ghm_sc_partials(pred_hbm, tgt_hbm, lw_hbm, out_hbm,
                     pa, pb, ta, tb, la, lb, stage, sem_a, sem_b):
    wid = lax.axis_index("s") * _NC + lax.axis_index("c")
    base = wid * _RPT

    zero = jnp.zeros((_L,), jnp.float32)
    bufs = ((pa, ta, la, sem_a), (pb, tb, lb, sem_b))

    def start(g, bs):
        r0 = base + g * _RCHUNK
        pltpu.async_copy(pred_hbm.at[pl.ds(r0, _RCHUNK), :], bs[0], bs[3])
        pltpu.async_copy(tgt_hbm.at[pl.ds(r0, _RCHUNK), :], bs[1], bs[3])
        pltpu.async_copy(lw_hbm.at[pl.ds(r0, _RCHUNK), :], bs[2], bs[3])

    def wait(bs):
        sl = pl.ds(0, _RCHUNK)
        pltpu.make_async_copy(pred_hbm.at[sl, :], bs[0], bs[3]).wait()
        pltpu.make_async_copy(tgt_hbm.at[sl, :], bs[1], bs[3]).wait()
        pltpu.make_async_copy(lw_hbm.at[sl, :], bs[2], bs[3]).wait()

    def process(bs, hist):
        pbuf, tbuf, lbuf, _ = bs

        def body(r, hist):
            cnts, sums = hist
            for c in range(_CV):
                sl = pl.ds(c * _L, _L)
                p = pbuf[r, sl]
                t = tbuf[r, sl]
                lwv = lbuf[r, sl]
                u = jnp.where(t > 0, -p, p)
                valid = lwv > 0
                uc = jnp.minimum(jnp.maximum(u, -_UCLAMP), _UCLAMP)
                x2 = uc * uc
                q = jnp.full((_L,), _SIG_C[0], jnp.float32)
                for cf in _SIG_C[1:]:
                    q = q * x2 + jnp.float32(cf)
                sig10 = uc * q + 5.0
                b_ = sig10.astype(jnp.int32)
                sel = jnp.where(valid, b_, _TRASH)
                e = jnp.exp(-jnp.abs(u))
                acc = jnp.full((_L,), _LOG1P_C[0], jnp.float32)
                for cf in _LOG1P_C[1:]:
                    acc = acc * e + jnp.float32(cf)
                bce = jnp.maximum(u, 0.0) + acc
                cnts = tuple(
                    cnts[b] + jnp.where(sel == b, 1.0, 0.0)
                    for b in range(_NBINS))
                sums = tuple(
                    sums[b] + jnp.where(sel == b, bce, 0.0)
                    for b in range(_NBINS))
            return (cnts, sums)

        return lax.fori_loop(0, _RCHUNK, body, hist)

    hist0 = (tuple(zero for _ in range(_NBINS)),
             tuple(zero for _ in range(_NBINS)))
    start(0, bufs[0])

    def pair_body(k, hist):
        g0 = 2 * k
        start(g0 + 1, bufs[1])
        wait(bufs[0])
        hist = process(bufs[0], hist)

        @pl.when(k < _G // 2 - 1)
        def _():
            start(g0 + 2, bufs[0])

        wait(bufs[1])
        return process(bufs[1], hist)

    cnts, sums = lax.fori_loop(0, _G // 2, pair_body, hist0)

    for b in range(_NBINS):
        stage[pl.ds(b * _L, _L)] = cnts[b]
        stage[pl.ds((_NBINS + b) * _L, _L)] = sums[b]
    pltpu.sync_copy(stage, out_hbm.at[pl.ds(wid * 2 * _NBINS * _L,
                                            2 * _NBINS * _L)])


_K = 4096.0


def _ghm_tc_body(pred_ref, tgt_ref, lw_ref, out_ref, sacc):
    i = pl.program_id(0)

    @pl.when(i == 0)
    def _():
        zeros = jnp.zeros((_RBLK, _COLS), jnp.float32)
        for b in range(_NBINS):
            sacc[b] = zeros

    p = pred_ref[...]
    t = tgt_ref[...].astype(jnp.float32)
    valid = lw_ref[...] > 0
    sig = jax.nn.sigmoid(p)
    g = jnp.abs(sig - t)
    b_ = jnp.minimum((g * jnp.float32(_NBINS)).astype(jnp.int32), _NBINS - 1)
    sel = jnp.where(valid, b_, _TRASH)
    bce = (jnp.maximum(p, 0.0) - p * t
           + jnp.log1p(jnp.exp(-jnp.abs(p))))
    val = bce + jnp.float32(_K)
    for b in range(_NBINS):
        sacc[b] += jnp.where(sel == b, val, 0.0)

    @pl.when(i == _GTC - 1)
    def _():
        for b in range(_NBINS):
            v = sacc[b]
            fl = jnp.floor(v * jnp.float32(1.0 / _K))
            out_ref[0, b] = jnp.sum(fl)
            out_ref[1, b] = jnp.sum(v - jnp.float32(_K) * fl)


_ghm_tc_partials = pl.pallas_call(
    _ghm_tc_body,
    grid=(_GTC,),
    in_specs=[
        pl.BlockSpec((_RBLK, _COLS), lambda i: (i + _RSC // _RBLK, 0)),
        pl.BlockSpec((_RBLK, _COLS), lambda i: (i + _RSC // _RBLK, 0)),
        pl.BlockSpec((_RBLK, _COLS), lambda i: (i + _RSC // _RBLK, 0)),
    ],
    out_specs=pl.BlockSpec(memory_space=pltpu.SMEM),
    out_shape=jax.ShapeDtypeStruct((2, _NBINS), jnp.float32),
    scratch_shapes=[
        pltpu.VMEM((_NBINS, _RBLK, _COLS), jnp.float32),
    ],
    compiler_params=pltpu.CompilerParams(
        dimension_semantics=("arbitrary",)),
)


def kernel(pred, target, label_weight):
    t32 = target.astype(jnp.int32)
    lw32 = label_weight.astype(jnp.int32)
    sc_partials = _ghm_sc_partials(pred, t32, lw32)
    tc_partials = _ghm_tc_partials(pred, t32, lw32)
    sc = sc_partials.reshape(_NW, 2, _NBINS, _L)
    cnts = jnp.sum(sc[:, 0], axis=(0, 2)) + tc_partials[0]
    sums = jnp.sum(sc[:, 1], axis=(0, 2)) + tc_partials[1]
    nz = cnts > 0.0
    n = jnp.sum(nz.astype(jnp.float32))
    contrib = jnp.sum(jnp.where(nz, sums / jnp.maximum(cnts, 1.0), 0.0))
    return jnp.where(n > 0.0, contrib / jnp.maximum(n, 1.0), 0.0)


# speedup vs baseline: 1.0699x; 1.0699x over previous
import functools

import jax
import jax.numpy as jnp
from jax import lax
from jax.experimental import pallas as pl
from jax.experimental.pallas import tpu as pltpu
from jax.experimental.pallas import tpu_sc as plsc

_NC = 2
_NS = 16
_NW = _NC * _NS
_L = 16

_ROWS = 131072
_COLS = 80
_RSC = 24576
_RTC = _ROWS - _RSC
_RPT = _RSC // _NW
_RCHUNK = 128
_G = _RPT // _RCHUNK
_CV = _COLS // _L

_RBLK = 512
_GTC = _RTC // _RBLK

_NBINS = 10
_TRASH = 12
_UCLAMP = 2.31

_LOG1P_C = (
    -0.006151544861495495, 0.03485012799501419, -0.09325294196605682,
    0.16582375764846802, -0.23982678353786469, 0.3315488398075104,
    -0.49983859062194824, 0.9999942779541016, 3.380091939675367e-08,
)
_SIG_C = (
    -5.25261384609621e-06, 0.00013854062126483768, -0.001915045897476375,
    0.020575666800141335, -0.2081817090511322, 2.499974250793457,
)

_mesh = plsc.VectorSubcoreMesh(core_axis_name="c", subcore_axis_name="s")


@functools.partial(
    pl.kernel,
    out_type=jax.ShapeDtypeStruct((_NW * 2 * _NBINS * _L,), jnp.float32),
    mesh=_mesh,
    scratch_types=[
        pltpu.VMEM((_RCHUNK, _COLS), jnp.float32),
        pltpu.VMEM((_RCHUNK, _COLS), jnp.float32),
        pltpu.VMEM((_RCHUNK, _COLS), jnp.int32),
        pltpu.VMEM((_RCHUNK, _COLS), jnp.int32),
        pltpu.VMEM((_RCHUNK, _COLS), jnp.int32),
        pltpu.VMEM((_RCHUNK, _COLS), jnp.int32),
        pltpu.VMEM((2 * _NBINS * _L,), jnp.float32),
        pltpu.SemaphoreType.DMA,
        pltpu.SemaphoreType.DMA,
    ],
    compiler_params=pltpu.CompilerParams(
        needs_layout_passes=False, use_tc_tiling_on_sc=True),
)
def _ghm_sc_partials(pred_hbm, tgt_hbm, lw_hbm, out_hbm,
                     pa, pb, ta, tb, la, lb, stage, sem_a, sem_b):
    wid = lax.axis_index("s") * _NC + lax.axis_index("c")
    base = wid * _RPT

    zero = jnp.zeros((_L,), jnp.float32)
    bufs = ((pa, ta, la, sem_a), (pb, tb, lb, sem_b))

    def start(g, bs):
        r0 = base + g * _RCHUNK
        pltpu.async_copy(pred_hbm.at[pl.ds(r0, _RCHUNK), :], bs[0], bs[3])
        pltpu.async_copy(tgt_hbm.at[pl.ds(r0, _RCHUNK), :], bs[1], bs[3])
        pltpu.async_copy(lw_hbm.at[pl.ds(r0, _RCHUNK), :], bs[2], bs[3])

    def wait(bs):
        sl = pl.ds(0, _RCHUNK)
        pltpu.make_async_copy(pred_hbm.at[sl, :], bs[0], bs[3]).wait()
        pltpu.make_async_copy(tgt_hbm.at[sl, :], bs[1], bs[3]).wait()
        pltpu.make_async_copy(lw_hbm.at[sl, :], bs[2], bs[3]).wait()

    def process(bs, hist):
        pbuf, tbuf, lbuf, _ = bs

        def body(r, hist):
            cnts, sums = hist
            for c in range(_CV):
                sl = pl.ds(c * _L, _L)
                p = pbuf[r, sl]
                t = tbuf[r, sl]
                lwv = lbuf[r, sl]
                u = jnp.where(t > 0, -p, p)
                valid = lwv > 0
                uc = jnp.minimum(jnp.maximum(u, -_UCLAMP), _UCLAMP)
                x2 = uc * uc
                q = jnp.full((_L,), _SIG_C[0], jnp.float32)
                for cf in _SIG_C[1:]:
                    q = q * x2 + jnp.float32(cf)
                sig10 = uc * q + 5.0
                b_ = sig10.astype(jnp.int32)
                sel = jnp.where(valid, b_, _TRASH)
                e = jnp.exp(-jnp.abs(u))
                acc = jnp.full((_L,), _LOG1P_C[0], jnp.float32)
                for cf in _LOG1P_C[1:]:
                    acc = acc * e + jnp.float32(cf)
                bce = jnp.maximum(u, 0.0) + acc
                cnts = tuple(
                    cnts[b] + jnp.where(sel == b, 1.0, 0.0)
                    for b in range(_NBINS))
                sums = tuple(
                    sums[b] + jnp.where(sel == b, bce, 0.0)
                    for b in range(_NBINS))
            return (cnts, sums)

        return lax.fori_loop(0, _RCHUNK, body, hist)

    hist0 = (tuple(zero for _ in range(_NBINS)),
             tuple(zero for _ in range(_NBINS)))
    start(0, bufs[0])

    def pair_body(k, hist):
        g0 = 2 * k
        start(g0 + 1, bufs[1])
        wait(bufs[0])
        hist = process(bufs[0], hist)

        @pl.when(k < _G // 2 - 1)
        def _():
            start(g0 + 2, bufs[0])

        wait(bufs[1])
        return process(bufs[1], hist)

    cnts, sums = lax.fori_loop(0, _G // 2, pair_body, hist0)

    for b in range(_NBINS):
        stage[pl.ds(b * _L, _L)] = cnts[b]
        stage[pl.ds((_NBINS + b) * _L, _L)] = sums[b]
    pltpu.sync_copy(stage, out_hbm.at[pl.ds(wid * 2 * _NBINS * _L,
                                            2 * _NBINS * _L)])


_K = 4096.0


def _ghm_tc_body(pred_ref, tgt_ref, lw_ref, out_ref, sacc):
    i = pl.program_id(0)

    @pl.when(i == 0)
    def _():
        zeros = jnp.zeros((_RBLK, _COLS), jnp.float32)
        for b in range(_NBINS):
            sacc[b] = zeros

    p = pred_ref[...]
    t = tgt_ref[...].astype(jnp.float32)
    valid = lw_ref[...] > 0
    sig = jax.nn.sigmoid(p)
    g = jnp.abs(sig - t)
    b_ = jnp.minimum((g * jnp.float32(_NBINS)).astype(jnp.int32), _NBINS - 1)
    sel = jnp.where(valid, b_, _TRASH)
    bce = (jnp.maximum(p, 0.0) - p * t
           + jnp.log1p(jnp.exp(-jnp.abs(p))))
    val = bce + jnp.float32(_K)
    for b in range(_NBINS):
        sacc[b] += jnp.where(sel == b, val, 0.0)

    @pl.when(i == _GTC - 1)
    def _():
        for b in range(_NBINS):
            v = sacc[b]
            fl = jnp.floor(v * jnp.float32(1.0 / _K))
            out_ref[0, b] = jnp.sum(fl)
            out_ref[1, b] = jnp.sum(v - jnp.float32(_K) * fl)


_ghm_tc_partials = pl.pallas_call(
    _ghm_tc_body,
    grid=(_GTC,),
    in_specs=[
        pl.BlockSpec((_RBLK, _COLS), lambda i: (i + _RSC // _RBLK, 0)),
        pl.BlockSpec((_RBLK, _COLS), lambda i: (i + _RSC // _RBLK, 0)),
        pl.BlockSpec((_RBLK, _COLS), lambda i: (i + _RSC // _RBLK, 0)),
    ],
    out_specs=pl.BlockSpec(memory_space=pltpu.SMEM),
    out_shape=jax.ShapeDtypeStruct((2, _NBINS), jnp.float32),
    scratch_shapes=[
        pltpu.VMEM((_NBINS, _RBLK, _COLS), jnp.float32),
    ],
    compiler_params=pltpu.CompilerParams(
        dimension_semantics=("arbitrary",)),
)


def kernel(pred, target, label_weight):
    t32 = target.astype(jnp.int32)
    lw32 = label_weight.astype(jnp.int32)
    sc_partials = _ghm_sc_partials(pred, t32, lw32)
    tc_partials = _ghm_tc_partials(pred, t32, lw32)
    sc = sc_partials.reshape(_NW, 2, _NBINS, _L)
    cnts = jnp.sum(sc[:, 0], axis=(0, 2)) + tc_partials[0]
    sums = jnp.sum(sc[:, 1], axis=(0, 2)) + tc_partials[1]
    nz = cnts > 0.0
    n = jnp.sum(nz.astype(jnp.float32))
    contrib = jnp.sum(jnp.where(nz, sums / jnp.maximum(cnts, 1.0), 0.0))
    return jnp.where(n > 0.0, contrib / jnp.maximum(n, 1.0), 0.0)
